# baseline (speedup 1.0000x reference)
import functools

import jax
import jax.numpy as jnp
from jax import lax
from jax.experimental import pallas as pl
from jax.experimental.pallas import tpu as pltpu
from jax.experimental.pallas import tpu_sc as plsc

N = 10000
E = 320000
D_IN = 128
D_H1 = 256
D_H2 = 512
D_OUT = 128

NC = 2
NT = 16
D = 128

NPAD = 10240
NPT = NPAD // NT
LANES = 128
GRP = 8 * LANES
G = 320
EP = G * GRP
NBATCH = 8 * G
ICH = 40

_NBP = 2048
_NB = 2000


def _vmesh():
    return plsc.VectorSubcoreMesh(core_axis_name="c", subcore_axis_name="s")


_SC_PARAMS = pltpu.CompilerParams(needs_layout_passes=False)


def _sc_degree(dst):
    EW = E // (NC * NT)

    @functools.partial(
        pl.kernel,
        out_type=jax.ShapeDtypeStruct((NC * NT * N,), jnp.float32),
        mesh=_vmesh(),
        compiler_params=_SC_PARAMS,
        scratch_types=[
            pltpu.VMEM((EW,), jnp.int32),
            pltpu.VMEM((N,), jnp.float32),
        ],
    )
    def deg_kernel(dst_hbm, out_hbm, idx_v, acc_v):
        c = lax.axis_index("c")
        s = lax.axis_index("s")
        wid = s * NC + c
        zeros16 = jnp.zeros((16,), jnp.float32)

        def zbody(i, carry):
            acc_v[pl.ds(i * 16, 16)] = zeros16
            return carry

        lax.fori_loop(0, N // 16, zbody, 0)
        pltpu.sync_copy(dst_hbm.at[pl.ds(wid * EW, EW)], idx_v)
        ones16 = jnp.ones((16,), jnp.float32)

        def body(k, carry):
            idx = idx_v[pl.ds(k * 16, 16)]
            plsc.addupdate_scatter(acc_v, [idx], ones16)
            return carry

        lax.fori_loop(0, EW // 16, body, 0)
        pltpu.sync_copy(acc_v, out_hbm.at[pl.ds(wid * N, N)])

    return deg_kernel(dst)


def _sc_propagate(g_flat, src3, dst3, split):
    ngrp = G if split else G // 2
    niter = (ngrp + NT - 1) // NT

    @functools.partial(
        pl.kernel,
        out_type=jax.ShapeDtypeStruct((2 * NPAD, D), jnp.float32),
        mesh=_vmesh(),
        compiler_params=_SC_PARAMS,
        scratch_types=[
            pltpu.VMEM((8, LANES), jnp.int32),
            pltpu.VMEM((8, LANES), jnp.int32),
            pltpu.VMEM((LANES, D), jnp.float32),
            pltpu.VMEM_SHARED((NPAD, D), jnp.float32),
            pltpu.SemaphoreType.DMA,
        ],
    )
    def prop_kernel(g_hbm, src_hbm, dst_hbm, out_hbm,
                    sidx_v, didx_v, rows_v, acc_s, sem0):
        c = lax.axis_index("c")
        s = lax.axis_index("s")
        goff = (c * NPAD if split else 0) + s * NPT
        pltpu.sync_copy(g_hbm.at[pl.ds(goff, NPT)],
                        acc_s.at[pl.ds(s * NPT, NPT)])
        grp0 = 0 if split else c * (G // 2)
        plsc.subcore_barrier()

        def body(i, carry):
            r = i * NT + s

            @pl.when(r < ngrp)
            def _():
                grp = grp0 + r
                pltpu.sync_copy(src_hbm.at[c, pl.ds(grp * 8, 8)], sidx_v)
                pltpu.sync_copy(dst_hbm.at[c, pl.ds(grp * 8, 8)], didx_v)
                for j in range(8):
                    pltpu.async_copy(g_hbm.at[sidx_v.at[j]], rows_v,
                                     sem0).wait()
                    pltpu.sync_copy(rows_v, acc_s.at[didx_v.at[j]],
                                    add=True)

            return carry

        lax.fori_loop(0, niter, body, 0)
        plsc.subcore_barrier()
        pltpu.sync_copy(acc_s.at[pl.ds(s * NPT, NPT)],
                        out_hbm.at[pl.ds(c * NPAD + s * NPT, NPT)])

    return prop_kernel(g_flat, src3, dst3).reshape(2, NPAD, D)


def _tc_prescale(x, dinv):
    def body(x_ref, dinv_ref, g_ref):
        g_ref[...] = x_ref[...] * dinv_ref[...]

    return pl.pallas_call(
        body,
        grid=(NPAD // _NBP,),
        in_specs=[pl.BlockSpec((_NBP, D_IN), lambda i: (i, 0)),
                  pl.BlockSpec((_NBP, 1), lambda i: (i, 0))],
        out_specs=pl.BlockSpec((_NBP, D_IN), lambda i: (i, 0)),
        out_shape=jax.ShapeDtypeStruct((NPAD, D_IN), jnp.float32),
    )(x, dinv)


def _tc_layer1(p, g0, dinv, W1, b1):
    def body(p_ref, g0_ref, dinv_ref, w_ref, b_ref, out_ref):
        s0 = p_ref[0] + p_ref[1] - g0_ref[...]
        z = jnp.dot(s0 * dinv_ref[...], w_ref[...],
                    preferred_element_type=jnp.float32) + b_ref[...]
        g1 = jnp.maximum(z, 0.0) * dinv_ref[...]
        out_ref[0] = g1[:, :D]
        out_ref[1] = g1[:, D:]

    return pl.pallas_call(
        body,
        grid=(NPAD // _NBP,),
        in_specs=[pl.BlockSpec((2, _NBP, D), lambda i: (0, i, 0)),
                  pl.BlockSpec((_NBP, D_IN), lambda i: (i, 0)),
                  pl.BlockSpec((_NBP, 1), lambda i: (i, 0)),
                  pl.BlockSpec((D_IN, D_H1), lambda i: (0, 0)),
                  pl.BlockSpec((1, D_H1), lambda i: (0, 0))],
        out_specs=pl.BlockSpec((2, _NBP, D), lambda i: (0, i, 0)),
        out_shape=jax.ShapeDtypeStruct((2, NPAD, D), jnp.float32),
    )(p, g0, dinv, W1, b1)


def _tc_layer23(p, dinv, W2, b2, W3):
    def body(p_ref, dinv_ref, w2_ref, b2_ref, w3_ref, out_ref):
        s1 = jnp.concatenate([p_ref[0], p_ref[1]], axis=1)
        z2 = jnp.dot(s1 * dinv_ref[...], w2_ref[...],
                     preferred_element_type=jnp.float32) + b2_ref[...]
        h2 = jnp.maximum(z2, 0.0)
        t = jnp.dot(h2, w3_ref[...], preferred_element_type=jnp.float32)
        out_ref[...] = t * dinv_ref[...]

    return pl.pallas_call(
        body,
        grid=(NPAD // _NBP,),
        in_specs=[pl.BlockSpec((2, _NBP, D), lambda i: (0, i, 0)),
                  pl.BlockSpec((_NBP, 1), lambda i: (i, 0)),
                  pl.BlockSpec((D_H1, D_H2), lambda i: (0, 0)),
                  pl.BlockSpec((1, D_H2), lambda i: (0, 0)),
                  pl.BlockSpec((D_H2, D_OUT), lambda i: (0, 0))],
        out_specs=pl.BlockSpec((_NBP, D_OUT), lambda i: (i, 0)),
        out_shape=jax.ShapeDtypeStruct((NPAD, D_OUT), jnp.float32),
    )(p, dinv, W2, b2, W3)


def _tc_final(p, g2, dinv, b3):
    def body(p_ref, g2_ref, dinv_ref, b_ref, out_ref):
        s2 = p_ref[0] + p_ref[1] - g2_ref[...]
        out_ref[...] = s2 * dinv_ref[...] + b_ref[...]

    return pl.pallas_call(
        body,
        grid=(N // _NB,),
        in_specs=[pl.BlockSpec((2, _NB, D), lambda i: (0, i, 0)),
                  pl.BlockSpec((_NB, D_OUT), lambda i: (i, 0)),
                  pl.BlockSpec((_NB, 1), lambda i: (i, 0)),
                  pl.BlockSpec((1, D_OUT), lambda i: (0, 0))],
        out_specs=pl.BlockSpec((_NB, D_OUT), lambda i: (i, 0)),
        out_shape=jax.ShapeDtypeStruct((N, D_OUT), jnp.float32),
    )(p, g2, dinv, b3)


def kernel(x, edge_index, W1, b1, W2, b2, W3, b3):
    src = edge_index[0]
    dst = edge_index[1]

    deg32 = _sc_degree(dst).reshape(NC * NT, N)
    dinv = lax.rsqrt(jnp.sum(deg32, axis=0) + 1.0)[:, None]
    dinv = jnp.pad(dinv, ((0, NPAD - N), (0, 0)))

    g0 = _tc_prescale(x, dinv)

    pad_dst = N + (jnp.arange(EP - E, dtype=dst.dtype) % (NPAD - N))
    src_p = jnp.concatenate(
        [src, jnp.zeros((EP - E,), src.dtype)]).reshape(NBATCH, LANES)
    dst_p = jnp.concatenate([dst, pad_dst]).reshape(NBATCH, LANES)
    src_dup = jnp.stack([src_p, src_p])
    dst_dup = jnp.stack([dst_p, dst_p])
    src_shift = jnp.stack([src_p, src_p + NPAD])

    p0 = _sc_propagate(g0, src_dup, dst_dup, split=False)
    g1 = _tc_layer1(p0, g0, dinv, W1, b1.reshape(1, -1))
    p1 = _sc_propagate(g1.reshape(2 * NPAD, D), src_shift, dst_dup,
                       split=True)
    g2 = _tc_layer23(p1, dinv, W2, b2.reshape(1, -1), W3)
    p2 = _sc_propagate(g2, src_dup, dst_dup, split=False)
    return _tc_final(p2, g2, dinv, b3.reshape(1, -1))


# speedup vs baseline: 1.9874x; 1.9874x over previous
import functools

import jax
import jax.numpy as jnp
from jax import lax
from jax.experimental import pallas as pl
from jax.experimental.pallas import tpu as pltpu
from jax.experimental.pallas import tpu_sc as plsc

N = 10000
E = 320000
D_IN = 128
D_H1 = 256
D_H2 = 512
D_OUT = 128

NC = 2
NT = 16
D = 128

NPAD = 10240
NPT = NPAD // NT
LANES = 128
GRP = 8 * LANES
G = 320
EP = G * GRP
NBATCH = 8 * G
ICH = 40

_NBP = 2048
_NB = 2000


def _vmesh():
    return plsc.VectorSubcoreMesh(core_axis_name="c", subcore_axis_name="s")


_SC_PARAMS = pltpu.CompilerParams(needs_layout_passes=False)


def _sc_degree(dst):
    EW = E // (NC * NT)

    @functools.partial(
        pl.kernel,
        out_type=jax.ShapeDtypeStruct((NC * NT * N,), jnp.float32),
        mesh=_vmesh(),
        compiler_params=_SC_PARAMS,
        scratch_types=[
            pltpu.VMEM((EW,), jnp.int32),
            pltpu.VMEM((N,), jnp.float32),
        ],
    )
    def deg_kernel(dst_hbm, out_hbm, idx_v, acc_v):
        c = lax.axis_index("c")
        s = lax.axis_index("s")
        wid = s * NC + c
        zeros16 = jnp.zeros((16,), jnp.float32)

        def zbody(i, carry):
            acc_v[pl.ds(i * 16, 16)] = zeros16
            return carry

        lax.fori_loop(0, N // 16, zbody, 0)
        pltpu.sync_copy(dst_hbm.at[pl.ds(wid * EW, EW)], idx_v)
        ones16 = jnp.ones((16,), jnp.float32)

        def body(k, carry):
            idx = idx_v[pl.ds(k * 16, 16)]
            plsc.addupdate_scatter(acc_v, [idx], ones16)
            return carry

        lax.fori_loop(0, EW // 16, body, 0)
        pltpu.sync_copy(acc_v, out_hbm.at[pl.ds(wid * N, N)])

    return deg_kernel(dst)


def _sc_propagate(g_flat, src3, dst3, split):
    ngrp = G if split else G // 2
    niter = ngrp // NT

    @functools.partial(
        pl.kernel,
        out_type=jax.ShapeDtypeStruct((2 * NPAD, D), jnp.float32),
        mesh=_vmesh(),
        compiler_params=_SC_PARAMS,
        scratch_types=[
            pltpu.VMEM((8, LANES), jnp.int32),
            pltpu.VMEM((8, LANES), jnp.int32),
            pltpu.VMEM((LANES, D), jnp.float32),
            pltpu.VMEM_SHARED((NPAD, D), jnp.float32),
            pltpu.SemaphoreType.DMA,
        ],
    )
    def prop_kernel(g_hbm, src_hbm, dst_hbm, out_hbm,
                    sidx_v, didx_v, rows_v, acc_s, sem0):
        c = lax.axis_index("c")
        s = lax.axis_index("s")
        goff = (c * NPAD if split else 0) + s * NPT
        pltpu.sync_copy(g_hbm.at[pl.ds(goff, NPT)],
                        acc_s.at[pl.ds(s * NPT, NPT)])
        grp0 = 0 if split else c * (G // 2)
        plsc.subcore_barrier()

        def body(i, carry):
            grp = grp0 + i * NT + s
            pltpu.sync_copy(src_hbm.at[c, pl.ds(grp * 8, 8)], sidx_v)
            pltpu.sync_copy(dst_hbm.at[c, pl.ds(grp * 8, 8)], didx_v)
            for j in range(8):
                pltpu.async_copy(g_hbm.at[sidx_v.at[j]], rows_v,
                                 sem0).wait()
                pltpu.sync_copy(rows_v, acc_s.at[didx_v.at[j]], add=True)
            return carry

        lax.fori_loop(0, niter, body, 0)
        plsc.subcore_barrier()
        pltpu.sync_copy(acc_s.at[pl.ds(s * NPT, NPT)],
                        out_hbm.at[pl.ds(c * NPAD + s * NPT, NPT)])

    return prop_kernel(g_flat, src3, dst3).reshape(2, NPAD, D)


def _tc_prescale(x, dinv):
    def body(x_ref, dinv_ref, g_ref):
        g_ref[...] = x_ref[...] * dinv_ref[...]

    return pl.pallas_call(
        body,
        grid=(NPAD // _NBP,),
        in_specs=[pl.BlockSpec((_NBP, D_IN), lambda i: (i, 0)),
                  pl.BlockSpec((_NBP, 1), lambda i: (i, 0))],
        out_specs=pl.BlockSpec((_NBP, D_IN), lambda i: (i, 0)),
        out_shape=jax.ShapeDtypeStruct((NPAD, D_IN), jnp.float32),
    )(x, dinv)


def _tc_layer1(p, g0, dinv, W1, b1):
    def body(p_ref, g0_ref, dinv_ref, w_ref, b_ref, out_ref):
        s0 = p_ref[0] + p_ref[1] - g0_ref[...]
        z = jnp.dot(s0 * dinv_ref[...], w_ref[...],
                    preferred_element_type=jnp.float32) + b_ref[...]
        g1 = jnp.maximum(z, 0.0) * dinv_ref[...]
        out_ref[0] = g1[:, :D]
        out_ref[1] = g1[:, D:]

    return pl.pallas_call(
        body,
        grid=(NPAD // _NBP,),
        in_specs=[pl.BlockSpec((2, _NBP, D), lambda i: (0, i, 0)),
                  pl.BlockSpec((_NBP, D_IN), lambda i: (i, 0)),
                  pl.BlockSpec((_NBP, 1), lambda i: (i, 0)),
                  pl.BlockSpec((D_IN, D_H1), lambda i: (0, 0)),
                  pl.BlockSpec((1, D_H1), lambda i: (0, 0))],
        out_specs=pl.BlockSpec((2, _NBP, D), lambda i: (0, i, 0)),
        out_shape=jax.ShapeDtypeStruct((2, NPAD, D), jnp.float32),
    )(p, g0, dinv, W1, b1)


def _tc_layer23(p, dinv, W2, b2, W3):
    def body(p_ref, dinv_ref, w2_ref, b2_ref, w3_ref, out_ref):
        s1 = jnp.concatenate([p_ref[0], p_ref[1]], axis=1)
        z2 = jnp.dot(s1 * dinv_ref[...], w2_ref[...],
                     preferred_element_type=jnp.float32) + b2_ref[...]
        h2 = jnp.maximum(z2, 0.0)
        t = jnp.dot(h2, w3_ref[...], preferred_element_type=jnp.float32)
        out_ref[...] = t * dinv_ref[...]

    return pl.pallas_call(
        body,
        grid=(NPAD // _NBP,),
        in_specs=[pl.BlockSpec((2, _NBP, D), lambda i: (0, i, 0)),
                  pl.BlockSpec((_NBP, 1), lambda i: (i, 0)),
                  pl.BlockSpec((D_H1, D_H2), lambda i: (0, 0)),
                  pl.BlockSpec((1, D_H2), lambda i: (0, 0)),
                  pl.BlockSpec((D_H2, D_OUT), lambda i: (0, 0))],
        out_specs=pl.BlockSpec((_NBP, D_OUT), lambda i: (i, 0)),
        out_shape=jax.ShapeDtypeStruct((NPAD, D_OUT), jnp.float32),
    )(p, dinv, W2, b2, W3)


def _tc_final(p, g2, dinv, b3):
    def body(p_ref, g2_ref, dinv_ref, b_ref, out_ref):
        s2 = p_ref[0] + p_ref[1] - g2_ref[...]
        out_ref[...] = s2 * dinv_ref[...] + b_ref[...]

    return pl.pallas_call(
        body,
        grid=(N // _NB,),
        in_specs=[pl.BlockSpec((2, _NB, D), lambda i: (0, i, 0)),
                  pl.BlockSpec((_NB, D_OUT), lambda i: (i, 0)),
                  pl.BlockSpec((_NB, 1), lambda i: (i, 0)),
                  pl.BlockSpec((1, D_OUT), lambda i: (0, 0))],
        out_specs=pl.BlockSpec((_NB, D_OUT), lambda i: (i, 0)),
        out_shape=jax.ShapeDtypeStruct((N, D_OUT), jnp.float32),
    )(p, g2, dinv, b3)


def kernel(x, edge_index, W1, b1, W2, b2, W3, b3):
    src = edge_index[0]
    dst = edge_index[1]

    deg32 = _sc_degree(dst).reshape(NC * NT, N)
    dinv = lax.rsqrt(jnp.sum(deg32, axis=0) + 1.0)[:, None]
    dinv = jnp.pad(dinv, ((0, NPAD - N), (0, 0)))

    g0 = _tc_prescale(x, dinv)

    pad_src = jnp.arange(EP - E, dtype=src.dtype) % N
    pad_dst = N + (jnp.arange(EP - E, dtype=dst.dtype) % (NPAD - N))
    src_p = jnp.concatenate([src, pad_src]).reshape(NBATCH, LANES)
    dst_p = jnp.concatenate([dst, pad_dst]).reshape(NBATCH, LANES)
    src_dup = jnp.stack([src_p, src_p])
    dst_dup = jnp.stack([dst_p, dst_p])
    src_shift = jnp.stack([src_p, src_p + NPAD])

    p0 = _sc_propagate(g0, src_dup, dst_dup, split=False)
    g1 = _tc_layer1(p0, g0, dinv, W1, b1.reshape(1, -1))
    p1 = _sc_propagate(g1.reshape(2 * NPAD, D), src_shift, dst_dup,
                       split=True)
    g2 = _tc_layer23(p1, dinv, W2, b2.reshape(1, -1), W3)
    p2 = _sc_propagate(g2, src_dup, dst_dup, split=False)
    return _tc_final(p2, g2, dinv, b3.reshape(1, -1))


# speedup vs baseline: 2.7277x; 1.3725x over previous
import functools

import jax
import jax.numpy as jnp
from jax import lax
from jax.experimental import pallas as pl
from jax.experimental.pallas import tpu as pltpu
from jax.experimental.pallas import tpu_sc as plsc

N = 10000
E = 320000
D_IN = 128
D_H1 = 256
D_H2 = 512
D_OUT = 128

NC = 2
NT = 16
D = 128

NPAD = 10240
NPT = NPAD // NT
LANES = 128
GRP = 8 * LANES
G = 320
EP = G * GRP
NBATCH = 8 * G
ICH = 40

_NBP = 2048
_NB = 2000


def _vmesh():
    return plsc.VectorSubcoreMesh(core_axis_name="c", subcore_axis_name="s")


_SC_PARAMS = pltpu.CompilerParams(needs_layout_passes=False)


def _sc_degree(dst):
    EW = E // (NC * NT)

    @functools.partial(
        pl.kernel,
        out_type=jax.ShapeDtypeStruct((NC * NT * N,), jnp.float32),
        mesh=_vmesh(),
        compiler_params=_SC_PARAMS,
        scratch_types=[
            pltpu.VMEM((EW,), jnp.int32),
            pltpu.VMEM((N,), jnp.float32),
        ],
    )
    def deg_kernel(dst_hbm, out_hbm, idx_v, acc_v):
        c = lax.axis_index("c")
        s = lax.axis_index("s")
        wid = s * NC + c
        zeros16 = jnp.zeros((16,), jnp.float32)

        def zbody(i, carry):
            acc_v[pl.ds(i * 16, 16)] = zeros16
            return carry

        lax.fori_loop(0, N // 16, zbody, 0)
        pltpu.sync_copy(dst_hbm.at[pl.ds(wid * EW, EW)], idx_v)
        ones16 = jnp.ones((16,), jnp.float32)

        def body(k, carry):
            idx = idx_v[pl.ds(k * 16, 16)]
            plsc.addupdate_scatter(acc_v, [idx], ones16)
            return carry

        lax.fori_loop(0, EW // 16, body, 0)
        pltpu.sync_copy(acc_v, out_hbm.at[pl.ds(wid * N, N)])

    return deg_kernel(dst)


def _sc_propagate(g_flat, src3, dst3, split):
    ngrp = G if split else G // 2
    niter = ngrp // NT

    @functools.partial(
        pl.kernel,
        out_type=jax.ShapeDtypeStruct((2 * NPAD, D), jnp.float32),
        mesh=_vmesh(),
        compiler_params=_SC_PARAMS,
        scratch_types=[
            pltpu.VMEM((8, LANES), jnp.int32),
            pltpu.VMEM((8, LANES), jnp.int32),
            pltpu.VMEM((LANES, D), jnp.float32),
            pltpu.VMEM((LANES, D), jnp.float32),
            pltpu.VMEM_SHARED((NPAD, D), jnp.float32),
            pltpu.SemaphoreType.DMA,
            pltpu.SemaphoreType.DMA,
        ],
    )
    def prop_kernel(g_hbm, src_hbm, dst_hbm, out_hbm,
                    sidx_v, didx_v, rows0_v, rows1_v, acc_s, sem0, sem1):
        rows = (rows0_v, rows1_v)
        sems = (sem0, sem1)
        c = lax.axis_index("c")
        s = lax.axis_index("s")
        goff = (c * NPAD if split else 0) + s * NPT
        pltpu.sync_copy(g_hbm.at[pl.ds(goff, NPT)],
                        acc_s.at[pl.ds(s * NPT, NPT)])
        grp0 = 0 if split else c * (G // 2)
        plsc.subcore_barrier()

        def body(i, carry):
            grp = grp0 + i * NT + s
            pltpu.sync_copy(src_hbm.at[c, pl.ds(grp * 8, 8)], sidx_v)
            pltpu.sync_copy(dst_hbm.at[c, pl.ds(grp * 8, 8)], didx_v)
            descs = [None, None]
            descs[0] = pltpu.async_copy(g_hbm.at[sidx_v.at[0]], rows[0],
                                        sems[0])
            for j in range(8):
                q = j % 2
                if j + 1 < 8:
                    descs[1 - q] = pltpu.async_copy(
                        g_hbm.at[sidx_v.at[j + 1]], rows[1 - q],
                        sems[1 - q])
                descs[q].wait()
                pltpu.sync_copy(rows[q], acc_s.at[didx_v.at[j]], add=True)
            return carry

        lax.fori_loop(0, niter, body, 0)
        plsc.subcore_barrier()
        pltpu.sync_copy(acc_s.at[pl.ds(s * NPT, NPT)],
                        out_hbm.at[pl.ds(c * NPAD + s * NPT, NPT)])

    return prop_kernel(g_flat, src3, dst3).reshape(2, NPAD, D)


def _tc_prescale(x, dinv):
    def body(x_ref, dinv_ref, g_ref):
        g_ref[...] = x_ref[...] * dinv_ref[...]

    return pl.pallas_call(
        body,
        grid=(NPAD // _NBP,),
        in_specs=[pl.BlockSpec((_NBP, D_IN), lambda i: (i, 0)),
                  pl.BlockSpec((_NBP, 1), lambda i: (i, 0))],
        out_specs=pl.BlockSpec((_NBP, D_IN), lambda i: (i, 0)),
        out_shape=jax.ShapeDtypeStruct((NPAD, D_IN), jnp.float32),
    )(x, dinv)


def _tc_layer1(p, g0, dinv, W1, b1):
    def body(p_ref, g0_ref, dinv_ref, w_ref, b_ref, out_ref):
        s0 = p_ref[0] + p_ref[1] - g0_ref[...]
        z = jnp.dot(s0 * dinv_ref[...], w_ref[...],
                    preferred_element_type=jnp.float32) + b_ref[...]
        g1 = jnp.maximum(z, 0.0) * dinv_ref[...]
        out_ref[0] = g1[:, :D]
        out_ref[1] = g1[:, D:]

    return pl.pallas_call(
        body,
        grid=(NPAD // _NBP,),
        in_specs=[pl.BlockSpec((2, _NBP, D), lambda i: (0, i, 0)),
                  pl.BlockSpec((_NBP, D_IN), lambda i: (i, 0)),
                  pl.BlockSpec((_NBP, 1), lambda i: (i, 0)),
                  pl.BlockSpec((D_IN, D_H1), lambda i: (0, 0)),
                  pl.BlockSpec((1, D_H1), lambda i: (0, 0))],
        out_specs=pl.BlockSpec((2, _NBP, D), lambda i: (0, i, 0)),
        out_shape=jax.ShapeDtypeStruct((2, NPAD, D), jnp.float32),
    )(p, g0, dinv, W1, b1)


def _tc_layer23(p, dinv, W2, b2, W3):
    def body(p_ref, dinv_ref, w2_ref, b2_ref, w3_ref, out_ref):
        s1 = jnp.concatenate([p_ref[0], p_ref[1]], axis=1)
        z2 = jnp.dot(s1 * dinv_ref[...], w2_ref[...],
                     preferred_element_type=jnp.float32) + b2_ref[...]
        h2 = jnp.maximum(z2, 0.0)
        t = jnp.dot(h2, w3_ref[...], preferred_element_type=jnp.float32)
        out_ref[...] = t * dinv_ref[...]

    return pl.pallas_call(
        body,
        grid=(NPAD // _NBP,),
        in_specs=[pl.BlockSpec((2, _NBP, D), lambda i: (0, i, 0)),
                  pl.BlockSpec((_NBP, 1), lambda i: (i, 0)),
                  pl.BlockSpec((D_H1, D_H2), lambda i: (0, 0)),
                  pl.BlockSpec((1, D_H2), lambda i: (0, 0)),
                  pl.BlockSpec((D_H2, D_OUT), lambda i: (0, 0))],
        out_specs=pl.BlockSpec((_NBP, D_OUT), lambda i: (i, 0)),
        out_shape=jax.ShapeDtypeStruct((NPAD, D_OUT), jnp.float32),
    )(p, dinv, W2, b2, W3)


def _tc_final(p, g2, dinv, b3):
    def body(p_ref, g2_ref, dinv_ref, b_ref, out_ref):
        s2 = p_ref[0] + p_ref[1] - g2_ref[...]
        out_ref[...] = s2 * dinv_ref[...] + b_ref[...]

    return pl.pallas_call(
        body,
        grid=(N // _NB,),
        in_specs=[pl.BlockSpec((2, _NB, D), lambda i: (0, i, 0)),
                  pl.BlockSpec((_NB, D_OUT), lambda i: (i, 0)),
                  pl.BlockSpec((_NB, 1), lambda i: (i, 0)),
                  pl.BlockSpec((1, D_OUT), lambda i: (0, 0))],
        out_specs=pl.BlockSpec((_NB, D_OUT), lambda i: (i, 0)),
        out_shape=jax.ShapeDtypeStruct((N, D_OUT), jnp.float32),
    )(p, g2, dinv, b3)


def kernel(x, edge_index, W1, b1, W2, b2, W3, b3):
    src = edge_index[0]
    dst = edge_index[1]

    deg32 = _sc_degree(dst).reshape(NC * NT, N)
    dinv = lax.rsqrt(jnp.sum(deg32, axis=0) + 1.0)[:, None]
    dinv = jnp.pad(dinv, ((0, NPAD - N), (0, 0)))

    g0 = _tc_prescale(x, dinv)

    pad_src = jnp.arange(EP - E, dtype=src.dtype) % N
    pad_dst = N + (jnp.arange(EP - E, dtype=dst.dtype) % (NPAD - N))
    src_p = jnp.concatenate([src, pad_src]).reshape(NBATCH, LANES)
    dst_p = jnp.concatenate([dst, pad_dst]).reshape(NBATCH, LANES)
    src_dup = jnp.stack([src_p, src_p])
    dst_dup = jnp.stack([dst_p, dst_p])
    src_shift = jnp.stack([src_p, src_p + NPAD])

    p0 = _sc_propagate(g0, src_dup, dst_dup, split=False)
    g1 = _tc_layer1(p0, g0, dinv, W1, b1.reshape(1, -1))
    p1 = _sc_propagate(g1.reshape(2 * NPAD, D), src_shift, dst_dup,
                       split=True)
    g2 = _tc_layer23(p1, dinv, W2, b2.reshape(1, -1), W3)
    p2 = _sc_propagate(g2, src_dup, dst_dup, split=False)
    return _tc_final(p2, g2, dinv, b3.reshape(1, -1))


# speedup vs baseline: 3.0185x; 1.1066x over previous
import functools

import jax
import jax.numpy as jnp
from jax import lax
from jax.experimental import pallas as pl
from jax.experimental.pallas import tpu as pltpu
from jax.experimental.pallas import tpu_sc as plsc

N = 10000
E = 320000
D_IN = 128
D_H1 = 256
D_H2 = 512
D_OUT = 128

NC = 2
NT = 16
D = 128

NPAD = 10240
NPT = NPAD // NT
LANES = 128
GRP = 8 * LANES
G = 320
EP = G * GRP
NBATCH = 8 * G
ICH = 40

_NBP = 2048
_NB = 2000


def _vmesh():
    return plsc.VectorSubcoreMesh(core_axis_name="c", subcore_axis_name="s")


_SC_PARAMS = pltpu.CompilerParams(needs_layout_passes=False)


def _sc_degree(dst):
    EW = E // (NC * NT)

    @functools.partial(
        pl.kernel,
        out_type=jax.ShapeDtypeStruct((NC * NT * N,), jnp.float32),
        mesh=_vmesh(),
        compiler_params=_SC_PARAMS,
        scratch_types=[
            pltpu.VMEM((EW,), jnp.int32),
            pltpu.VMEM((N,), jnp.float32),
        ],
    )
    def deg_kernel(dst_hbm, out_hbm, idx_v, acc_v):
        c = lax.axis_index("c")
        s = lax.axis_index("s")
        wid = s * NC + c
        zeros16 = jnp.zeros((16,), jnp.float32)

        def zbody(i, carry):
            acc_v[pl.ds(i * 16, 16)] = zeros16
            return carry

        lax.fori_loop(0, N // 16, zbody, 0)
        pltpu.sync_copy(dst_hbm.at[pl.ds(wid * EW, EW)], idx_v)
        ones16 = jnp.ones((16,), jnp.float32)

        def body(k, carry):
            idx = idx_v[pl.ds(k * 16, 16)]
            plsc.addupdate_scatter(acc_v, [idx], ones16)
            return carry

        lax.fori_loop(0, EW // 16, body, 0)
        pltpu.sync_copy(acc_v, out_hbm.at[pl.ds(wid * N, N)])

    return deg_kernel(dst)


def _sc_propagate(g_flat, idx3, split):
    ngrp = G if split else G // 2
    niter = ngrp // (2 * NT)

    @functools.partial(
        pl.kernel,
        out_type=jax.ShapeDtypeStruct((2 * NPAD, D), jnp.float32),
        mesh=_vmesh(),
        compiler_params=_SC_PARAMS,
        scratch_types=[
            pltpu.VMEM((32, LANES), jnp.int32),
            pltpu.VMEM((LANES, D), jnp.float32),
            pltpu.VMEM((LANES, D), jnp.float32),
            pltpu.VMEM_SHARED((NPAD, D), jnp.float32),
            pltpu.SemaphoreType.DMA,
            pltpu.SemaphoreType.DMA,
        ],
    )
    def prop_kernel(g_hbm, idx_hbm, out_hbm,
                    idx_v, rows0_v, rows1_v, acc_s, sem0, sem1):
        rows = (rows0_v, rows1_v)
        sems = (sem0, sem1)
        c = lax.axis_index("c")
        s = lax.axis_index("s")
        goff = (c * NPAD if split else 0) + s * NPT
        pltpu.sync_copy(g_hbm.at[pl.ds(goff, NPT)],
                        acc_s.at[pl.ds(s * NPT, NPT)])
        grp0 = 0 if split else c * (G // 2)
        plsc.subcore_barrier()

        def body(i, carry):
            grp = grp0 + (i * NT + s) * 2
            pltpu.sync_copy(idx_hbm.at[c, pl.ds(grp * 16, 32)], idx_v)
            descs = [None, None]
            descs[0] = pltpu.async_copy(g_hbm.at[idx_v.at[0]], rows[0],
                                        sems[0])
            for j in range(16):
                q = j % 2
                half, jj = divmod(j, 8)
                if j + 1 < 16:
                    h2, j2 = divmod(j + 1, 8)
                    descs[1 - q] = pltpu.async_copy(
                        g_hbm.at[idx_v.at[h2 * 16 + j2]], rows[1 - q],
                        sems[1 - q])
                descs[q].wait()
                pltpu.sync_copy(rows[q],
                                acc_s.at[idx_v.at[half * 16 + 8 + jj]],
                                add=True)
            return carry

        lax.fori_loop(0, niter, body, 0)
        plsc.subcore_barrier()
        pltpu.sync_copy(acc_s.at[pl.ds(s * NPT, NPT)],
                        out_hbm.at[pl.ds(c * NPAD + s * NPT, NPT)])

    return prop_kernel(g_flat, idx3).reshape(2, NPAD, D)


def _tc_prescale(x, dinv):
    def body(x_ref, dinv_ref, g_ref):
        g_ref[...] = x_ref[...] * dinv_ref[...]

    return pl.pallas_call(
        body,
        grid=(NPAD // _NBP,),
        in_specs=[pl.BlockSpec((_NBP, D_IN), lambda i: (i, 0)),
                  pl.BlockSpec((_NBP, 1), lambda i: (i, 0))],
        out_specs=pl.BlockSpec((_NBP, D_IN), lambda i: (i, 0)),
        out_shape=jax.ShapeDtypeStruct((NPAD, D_IN), jnp.float32),
    )(x, dinv)


def _tc_layer1(p, g0, dinv, W1, b1):
    def body(p_ref, g0_ref, dinv_ref, w_ref, b_ref, out_ref):
        s0 = p_ref[0] + p_ref[1] - g0_ref[...]
        z = jnp.dot(s0 * dinv_ref[...], w_ref[...],
                    preferred_element_type=jnp.float32) + b_ref[...]
        g1 = jnp.maximum(z, 0.0) * dinv_ref[...]
        out_ref[0] = g1[:, :D]
        out_ref[1] = g1[:, D:]

    return pl.pallas_call(
        body,
        grid=(NPAD // _NBP,),
        in_specs=[pl.BlockSpec((2, _NBP, D), lambda i: (0, i, 0)),
                  pl.BlockSpec((_NBP, D_IN), lambda i: (i, 0)),
                  pl.BlockSpec((_NBP, 1), lambda i: (i, 0)),
                  pl.BlockSpec((D_IN, D_H1), lambda i: (0, 0)),
                  pl.BlockSpec((1, D_H1), lambda i: (0, 0))],
        out_specs=pl.BlockSpec((2, _NBP, D), lambda i: (0, i, 0)),
        out_shape=jax.ShapeDtypeStruct((2, NPAD, D), jnp.float32),
    )(p, g0, dinv, W1, b1)


def _tc_layer23(p, dinv, W2, b2, W3):
    def body(p_ref, dinv_ref, w2_ref, b2_ref, w3_ref, out_ref):
        s1 = jnp.concatenate([p_ref[0], p_ref[1]], axis=1)
        z2 = jnp.dot(s1 * dinv_ref[...], w2_ref[...],
                     preferred_element_type=jnp.float32) + b2_ref[...]
        h2 = jnp.maximum(z2, 0.0)
        t = jnp.dot(h2, w3_ref[...], preferred_element_type=jnp.float32)
        out_ref[...] = t * dinv_ref[...]

    return pl.pallas_call(
        body,
        grid=(NPAD // _NBP,),
        in_specs=[pl.BlockSpec((2, _NBP, D), lambda i: (0, i, 0)),
                  pl.BlockSpec((_NBP, 1), lambda i: (i, 0)),
                  pl.BlockSpec((D_H1, D_H2), lambda i: (0, 0)),
                  pl.BlockSpec((1, D_H2), lambda i: (0, 0)),
                  pl.BlockSpec((D_H2, D_OUT), lambda i: (0, 0))],
        out_specs=pl.BlockSpec((_NBP, D_OUT), lambda i: (i, 0)),
        out_shape=jax.ShapeDtypeStruct((NPAD, D_OUT), jnp.float32),
    )(p, dinv, W2, b2, W3)


def _tc_final(p, g2, dinv, b3):
    def body(p_ref, g2_ref, dinv_ref, b_ref, out_ref):
        s2 = p_ref[0] + p_ref[1] - g2_ref[...]
        out_ref[...] = s2 * dinv_ref[...] + b_ref[...]

    return pl.pallas_call(
        body,
        grid=(N // _NB,),
        in_specs=[pl.BlockSpec((2, _NB, D), lambda i: (0, i, 0)),
                  pl.BlockSpec((_NB, D_OUT), lambda i: (i, 0)),
                  pl.BlockSpec((_NB, 1), lambda i: (i, 0)),
                  pl.BlockSpec((1, D_OUT), lambda i: (0, 0))],
        out_specs=pl.BlockSpec((_NB, D_OUT), lambda i: (i, 0)),
        out_shape=jax.ShapeDtypeStruct((N, D_OUT), jnp.float32),
    )(p, g2, dinv, b3)


def kernel(x, edge_index, W1, b1, W2, b2, W3, b3):
    src = edge_index[0]
    dst = edge_index[1]

    deg32 = _sc_degree(dst).reshape(NC * NT, N)
    dinv = lax.rsqrt(jnp.sum(deg32, axis=0) + 1.0)[:, None]
    dinv = jnp.pad(dinv, ((0, NPAD - N), (0, 0)))

    g0 = _tc_prescale(x, dinv)

    pad_src = jnp.arange(EP - E, dtype=src.dtype) % N
    pad_dst = N + (jnp.arange(EP - E, dtype=dst.dtype) % (NPAD - N))
    src_g = jnp.concatenate([src, pad_src]).reshape(G, 8, LANES)
    dst_g = jnp.concatenate([dst, pad_dst]).reshape(G, 8, LANES)
    pack = jnp.concatenate([src_g, dst_g], axis=1).reshape(16 * G, LANES)
    pack_sh = jnp.concatenate([src_g + NPAD, dst_g],
                              axis=1).reshape(16 * G, LANES)
    idx_dup = jnp.stack([pack, pack])
    idx_shift = jnp.stack([pack, pack_sh])

    p0 = _sc_propagate(g0, idx_dup, split=False)
    g1 = _tc_layer1(p0, g0, dinv, W1, b1.reshape(1, -1))
    p1 = _sc_propagate(g1.reshape(2 * NPAD, D), idx_shift, split=True)
    g2 = _tc_layer23(p1, dinv, W2, b2.reshape(1, -1), W3)
    p2 = _sc_propagate(g2, idx_dup, split=False)
    return _tc_final(p2, g2, dinv, b3.reshape(1, -1))


# speedup vs baseline: 3.0710x; 1.0174x over previous
import functools

import jax
import jax.numpy as jnp
from jax import lax
from jax.experimental import pallas as pl
from jax.experimental.pallas import tpu as pltpu
from jax.experimental.pallas import tpu_sc as plsc

N = 10000
E = 320000
D_IN = 128
D_H1 = 256
D_H2 = 512
D_OUT = 128

NC = 2
NT = 16
D = 128

NPAD = 10240
NPT = NPAD // NT
LANES = 128
GRP = 8 * LANES
G = 320
EP = G * GRP
NBATCH = 8 * G
ICH = 40

_NBP = 2048
_NB = 2000


def _vmesh():
    return plsc.VectorSubcoreMesh(core_axis_name="c", subcore_axis_name="s")


_SC_PARAMS = pltpu.CompilerParams(needs_layout_passes=False)


def _sc_degree(dst):
    EW = E // (NC * NT)

    @functools.partial(
        pl.kernel,
        out_type=jax.ShapeDtypeStruct((NC * NT * N,), jnp.float32),
        mesh=_vmesh(),
        compiler_params=_SC_PARAMS,
        scratch_types=[
            pltpu.VMEM((EW,), jnp.int32),
            pltpu.VMEM((N,), jnp.float32),
        ],
    )
    def deg_kernel(dst_hbm, out_hbm, idx_v, acc_v):
        c = lax.axis_index("c")
        s = lax.axis_index("s")
        wid = s * NC + c
        zeros16 = jnp.zeros((16,), jnp.float32)

        def zbody(i, carry):
            acc_v[pl.ds(i * 16, 16)] = zeros16
            return carry

        lax.fori_loop(0, N // 16, zbody, 0)
        pltpu.sync_copy(dst_hbm.at[pl.ds(wid * EW, EW)], idx_v)
        ones16 = jnp.ones((16,), jnp.float32)

        def body(k, carry):
            idx = idx_v[pl.ds(k * 16, 16)]
            plsc.addupdate_scatter(acc_v, [idx], ones16)
            return carry

        lax.fori_loop(0, EW // 16, body, 0)
        pltpu.sync_copy(acc_v, out_hbm.at[pl.ds(wid * N, N)])

    return deg_kernel(dst)


def _sc_propagate(g_flat, idx3, split):
    ngrp = G if split else G // 2
    GB = 4 if split else 2
    niter = ngrp // (GB * NT)

    @functools.partial(
        pl.kernel,
        out_type=jax.ShapeDtypeStruct((2 * NPAD, D), jnp.float32),
        mesh=_vmesh(),
        compiler_params=_SC_PARAMS,
        scratch_types=[
            pltpu.VMEM((16 * GB, LANES), jnp.int32),
            pltpu.VMEM((LANES, D), jnp.float32),
            pltpu.VMEM((LANES, D), jnp.float32),
            pltpu.VMEM_SHARED((NPAD, D), jnp.float32),
            pltpu.SemaphoreType.DMA,
            pltpu.SemaphoreType.DMA,
        ],
    )
    def prop_kernel(g_hbm, idx_hbm, out_hbm,
                    idx_v, rows0_v, rows1_v, acc_s, sem0, sem1):
        rows = (rows0_v, rows1_v)
        sems = (sem0, sem1)
        c = lax.axis_index("c")
        s = lax.axis_index("s")
        goff = (c * NPAD if split else 0) + s * NPT
        pltpu.sync_copy(g_hbm.at[pl.ds(goff, NPT)],
                        acc_s.at[pl.ds(s * NPT, NPT)])
        grp0 = 0 if split else c * (G // 2)
        plsc.subcore_barrier()

        nbb = 8 * GB

        def body(i, carry):
            grp = grp0 + (i * NT + s) * GB
            pltpu.sync_copy(idx_hbm.at[c, pl.ds(grp * 16, 16 * GB)], idx_v)
            descs = [None, None]
            descs[0] = pltpu.async_copy(g_hbm.at[idx_v.at[0]], rows[0],
                                        sems[0])
            for j in range(nbb):
                q = j % 2
                half, jj = divmod(j, 8)
                if j + 1 < nbb:
                    h2, j2 = divmod(j + 1, 8)
                    descs[1 - q] = pltpu.async_copy(
                        g_hbm.at[idx_v.at[h2 * 16 + j2]], rows[1 - q],
                        sems[1 - q])
                descs[q].wait()
                pltpu.sync_copy(rows[q],
                                acc_s.at[idx_v.at[half * 16 + 8 + jj]],
                                add=True)
            return carry

        lax.fori_loop(0, niter, body, 0)
        plsc.subcore_barrier()
        pltpu.sync_copy(acc_s.at[pl.ds(s * NPT, NPT)],
                        out_hbm.at[pl.ds(c * NPAD + s * NPT, NPT)])

    return prop_kernel(g_flat, idx3).reshape(2, NPAD, D)


def _tc_prescale(x, dinv):
    def body(x_ref, dinv_ref, g_ref):
        g_ref[...] = x_ref[...] * dinv_ref[...]

    return pl.pallas_call(
        body,
        grid=(NPAD // _NBP,),
        in_specs=[pl.BlockSpec((_NBP, D_IN), lambda i: (i, 0)),
                  pl.BlockSpec((_NBP, 1), lambda i: (i, 0))],
        out_specs=pl.BlockSpec((_NBP, D_IN), lambda i: (i, 0)),
        out_shape=jax.ShapeDtypeStruct((NPAD, D_IN), jnp.float32),
    )(x, dinv)


def _tc_layer1(p, g0, dinv, W1, b1):
    def body(p_ref, g0_ref, dinv_ref, w_ref, b_ref, out_ref):
        s0 = p_ref[0] + p_ref[1] - g0_ref[...]
        z = jnp.dot(s0 * dinv_ref[...], w_ref[...],
                    preferred_element_type=jnp.float32) + b_ref[...]
        g1 = jnp.maximum(z, 0.0) * dinv_ref[...]
        out_ref[0] = g1[:, :D]
        out_ref[1] = g1[:, D:]

    return pl.pallas_call(
        body,
        grid=(NPAD // _NBP,),
        in_specs=[pl.BlockSpec((2, _NBP, D), lambda i: (0, i, 0)),
                  pl.BlockSpec((_NBP, D_IN), lambda i: (i, 0)),
                  pl.BlockSpec((_NBP, 1), lambda i: (i, 0)),
                  pl.BlockSpec((D_IN, D_H1), lambda i: (0, 0)),
                  pl.BlockSpec((1, D_H1), lambda i: (0, 0))],
        out_specs=pl.BlockSpec((2, _NBP, D), lambda i: (0, i, 0)),
        out_shape=jax.ShapeDtypeStruct((2, NPAD, D), jnp.float32),
    )(p, g0, dinv, W1, b1)


def _tc_layer23(p, dinv, W2, b2, W3):
    def body(p_ref, dinv_ref, w2_ref, b2_ref, w3_ref, out_ref):
        s1 = jnp.concatenate([p_ref[0], p_ref[1]], axis=1)
        z2 = jnp.dot(s1 * dinv_ref[...], w2_ref[...],
                     preferred_element_type=jnp.float32) + b2_ref[...]
        h2 = jnp.maximum(z2, 0.0)
        t = jnp.dot(h2, w3_ref[...], preferred_element_type=jnp.float32)
        out_ref[...] = t * dinv_ref[...]

    return pl.pallas_call(
        body,
        grid=(NPAD // _NBP,),
        in_specs=[pl.BlockSpec((2, _NBP, D), lambda i: (0, i, 0)),
                  pl.BlockSpec((_NBP, 1), lambda i: (i, 0)),
                  pl.BlockSpec((D_H1, D_H2), lambda i: (0, 0)),
                  pl.BlockSpec((1, D_H2), lambda i: (0, 0)),
                  pl.BlockSpec((D_H2, D_OUT), lambda i: (0, 0))],
        out_specs=pl.BlockSpec((_NBP, D_OUT), lambda i: (i, 0)),
        out_shape=jax.ShapeDtypeStruct((NPAD, D_OUT), jnp.float32),
    )(p, dinv, W2, b2, W3)


def _tc_final(p, g2, dinv, b3):
    def body(p_ref, g2_ref, dinv_ref, b_ref, out_ref):
        s2 = p_ref[0] + p_ref[1] - g2_ref[...]
        out_ref[...] = s2 * dinv_ref[...] + b_ref[...]

    return pl.pallas_call(
        body,
        grid=(N // _NB,),
        in_specs=[pl.BlockSpec((2, _NB, D), lambda i: (0, i, 0)),
                  pl.BlockSpec((_NB, D_OUT), lambda i: (i, 0)),
                  pl.BlockSpec((_NB, 1), lambda i: (i, 0)),
                  pl.BlockSpec((1, D_OUT), lambda i: (0, 0))],
        out_specs=pl.BlockSpec((_NB, D_OUT), lambda i: (i, 0)),
        out_shape=jax.ShapeDtypeStruct((N, D_OUT), jnp.float32),
    )(p, g2, dinv, b3)


def kernel(x, edge_index, W1, b1, W2, b2, W3, b3):
    src = edge_index[0]
    dst = edge_index[1]

    deg32 = _sc_degree(dst).reshape(NC * NT, N)
    dinv = lax.rsqrt(jnp.sum(deg32, axis=0) + 1.0)[:, None]
    dinv = jnp.pad(dinv, ((0, NPAD - N), (0, 0)))

    g0 = _tc_prescale(x, dinv)

    pad_src = jnp.arange(EP - E, dtype=src.dtype) % N
    pad_dst = N + (jnp.arange(EP - E, dtype=dst.dtype) % (NPAD - N))
    src_g = jnp.concatenate([src, pad_src]).reshape(G, 8, LANES)
    dst_g = jnp.concatenate([dst, pad_dst]).reshape(G, 8, LANES)
    pack = jnp.concatenate([src_g, dst_g], axis=1).reshape(16 * G, LANES)
    pack_sh = jnp.concatenate([src_g + NPAD, dst_g],
                              axis=1).reshape(16 * G, LANES)
    idx_dup = jnp.stack([pack, pack])
    idx_shift = jnp.stack([pack, pack_sh])

    p0 = _sc_propagate(g0, idx_dup, split=False)
    g1 = _tc_layer1(p0, g0, dinv, W1, b1.reshape(1, -1))
    p1 = _sc_propagate(g1.reshape(2 * NPAD, D), idx_shift, split=True)
    g2 = _tc_layer23(p1, dinv, W2, b2.reshape(1, -1), W3)
    p2 = _sc_propagate(g2, idx_dup, split=False)
    return _tc_final(p2, g2, dinv, b3.reshape(1, -1))


# speedup vs baseline: 3.0710x; 1.0000x over previous
import functools

import jax
import jax.numpy as jnp
from jax import lax
from jax.experimental import pallas as pl
from jax.experimental.pallas import tpu as pltpu
from jax.experimental.pallas import tpu_sc as plsc

N = 10000
E = 320000
D_IN = 128
D_H1 = 256
D_H2 = 512
D_OUT = 128

NC = 2
NT = 16
D = 128

NPAD = 10240
NPT = NPAD // NT
LANES = 128
GRP = 8 * LANES
G = 320
EP = G * GRP
NBATCH = 8 * G

_NBP = 2048
_NB = 2000


def _vmesh():
    return plsc.VectorSubcoreMesh(core_axis_name="c", subcore_axis_name="s")


_SC_PARAMS = pltpu.CompilerParams(needs_layout_passes=False)


def _sc_degree(dst):
    EW = E // (NC * NT)

    @functools.partial(
        pl.kernel,
        out_type=jax.ShapeDtypeStruct((NC * NT * N,), jnp.float32),
        mesh=_vmesh(),
        compiler_params=_SC_PARAMS,
        scratch_types=[
            pltpu.VMEM((EW,), jnp.int32),
            pltpu.VMEM((N,), jnp.float32),
        ],
    )
    def deg_kernel(dst_hbm, out_hbm, idx_v, acc_v):
        c = lax.axis_index("c")
        s = lax.axis_index("s")
        wid = s * NC + c
        zeros16 = jnp.zeros((16,), jnp.float32)

        def zbody(i, carry):
            acc_v[pl.ds(i * 16, 16)] = zeros16
            return carry

        lax.fori_loop(0, N // 16, zbody, 0)
        pltpu.sync_copy(dst_hbm.at[pl.ds(wid * EW, EW)], idx_v)
        ones16 = jnp.ones((16,), jnp.float32)

        def body(k, carry):
            idx = idx_v[pl.ds(k * 16, 16)]
            plsc.addupdate_scatter(acc_v, [idx], ones16)
            return carry

        lax.fori_loop(0, EW // 16, body, 0)
        pltpu.sync_copy(acc_v, out_hbm.at[pl.ds(wid * N, N)])

    return deg_kernel(dst)


def _sc_propagate(g_flat, idx3, split):
    ngrp = G if split else G // 2
    GB = 4 if split else 2
    niter = ngrp // (GB * NT)

    @functools.partial(
        pl.kernel,
        out_type=jax.ShapeDtypeStruct((2 * NPAD, D), jnp.float32),
        mesh=_vmesh(),
        compiler_params=_SC_PARAMS,
        scratch_types=[
            pltpu.VMEM((16 * GB, LANES), jnp.int32),
            pltpu.VMEM((LANES, D), jnp.float32),
            pltpu.VMEM((LANES, D), jnp.float32),
            pltpu.VMEM_SHARED((NPAD, D), jnp.float32),
            pltpu.SemaphoreType.DMA,
            pltpu.SemaphoreType.DMA,
        ],
    )
    def prop_kernel(g_hbm, idx_hbm, out_hbm,
                    idx_v, rows0_v, rows1_v, acc_s, sem0, sem1):
        rows = (rows0_v, rows1_v)
        sems = (sem0, sem1)
        c = lax.axis_index("c")
        s = lax.axis_index("s")
        goff = (c * NPAD if split else 0) + s * NPT
        pltpu.sync_copy(g_hbm.at[pl.ds(goff, NPT)],
                        acc_s.at[pl.ds(s * NPT, NPT)])
        grp0 = 0 if split else c * (G // 2)
        plsc.subcore_barrier()

        nbb = 8 * GB

        def body(i, carry):
            grp = grp0 + (i * NT + s) * GB
            pltpu.sync_copy(idx_hbm.at[c, pl.ds(grp * 16, 16 * GB)], idx_v)
            descs = [None, None]
            descs[0] = pltpu.async_copy(g_hbm.at[idx_v.at[0]], rows[0],
                                        sems[0])
            for j in range(nbb):
                q = j % 2
                half, jj = divmod(j, 8)
                if j + 1 < nbb:
                    h2, j2 = divmod(j + 1, 8)
                    descs[1 - q] = pltpu.async_copy(
                        g_hbm.at[idx_v.at[h2 * 16 + j2]], rows[1 - q],
                        sems[1 - q])
                descs[q].wait()
                pltpu.sync_copy(rows[q],
                                acc_s.at[idx_v.at[half * 16 + 8 + jj]],
                                add=True)
            return carry

        lax.fori_loop(0, niter, body, 0)
        plsc.subcore_barrier()
        pltpu.sync_copy(acc_s.at[pl.ds(s * NPT, NPT)],
                        out_hbm.at[pl.ds(c * NPAD + s * NPT, NPT)])

    return prop_kernel(g_flat, idx3).reshape(2, NPAD, D)


def _tc_prescale(x, dinv):
    def body(x_ref, dinv_ref, g_ref):
        g_ref[...] = x_ref[...] * dinv_ref[...]

    return pl.pallas_call(
        body,
        grid=(NPAD // _NBP,),
        in_specs=[pl.BlockSpec((_NBP, D_IN), lambda i: (i, 0)),
                  pl.BlockSpec((_NBP, 1), lambda i: (i, 0))],
        out_specs=pl.BlockSpec((_NBP, D_IN), lambda i: (i, 0)),
        out_shape=jax.ShapeDtypeStruct((NPAD, D_IN), jnp.float32),
    )(x, dinv)


def _tc_layer1(p, g0, dinv, W1, b1):
    def body(p_ref, g0_ref, dinv_ref, w_ref, b_ref, out_ref):
        s0 = p_ref[0] + p_ref[1] - g0_ref[...]
        z = jnp.dot(s0 * dinv_ref[...], w_ref[...],
                    preferred_element_type=jnp.float32) + b_ref[...]
        g1 = jnp.maximum(z, 0.0) * dinv_ref[...]
        out_ref[0] = g1[:, :D]
        out_ref[1] = g1[:, D:]

    return pl.pallas_call(
        body,
        grid=(NPAD // _NBP,),
        in_specs=[pl.BlockSpec((2, _NBP, D), lambda i: (0, i, 0)),
                  pl.BlockSpec((_NBP, D_IN), lambda i: (i, 0)),
                  pl.BlockSpec((_NBP, 1), lambda i: (i, 0)),
                  pl.BlockSpec((D_IN, D_H1), lambda i: (0, 0)),
                  pl.BlockSpec((1, D_H1), lambda i: (0, 0))],
        out_specs=pl.BlockSpec((2, _NBP, D), lambda i: (0, i, 0)),
        out_shape=jax.ShapeDtypeStruct((2, NPAD, D), jnp.float32),
    )(p, g0, dinv, W1, b1)


def _tc_layer23(p, dinv, W2, b2, W3):
    def body(p_ref, dinv_ref, w2_ref, b2_ref, w3_ref, out_ref):
        s1 = jnp.concatenate([p_ref[0], p_ref[1]], axis=1)
        z2 = jnp.dot(s1 * dinv_ref[...], w2_ref[...],
                     preferred_element_type=jnp.float32) + b2_ref[...]
        h2 = jnp.maximum(z2, 0.0)
        t = jnp.dot(h2, w3_ref[...], preferred_element_type=jnp.float32)
        out_ref[...] = t * dinv_ref[...]

    return pl.pallas_call(
        body,
        grid=(NPAD // _NBP,),
        in_specs=[pl.BlockSpec((2, _NBP, D), lambda i: (0, i, 0)),
                  pl.BlockSpec((_NBP, 1), lambda i: (i, 0)),
                  pl.BlockSpec((D_H1, D_H2), lambda i: (0, 0)),
                  pl.BlockSpec((1, D_H2), lambda i: (0, 0)),
                  pl.BlockSpec((D_H2, D_OUT), lambda i: (0, 0))],
        out_specs=pl.BlockSpec((_NBP, D_OUT), lambda i: (i, 0)),
        out_shape=jax.ShapeDtypeStruct((NPAD, D_OUT), jnp.float32),
    )(p, dinv, W2, b2, W3)


def _tc_final(p, g2, dinv, b3):
    def body(p_ref, g2_ref, dinv_ref, b_ref, out_ref):
        s2 = p_ref[0] + p_ref[1] - g2_ref[...]
        out_ref[...] = s2 * dinv_ref[...] + b_ref[...]

    return pl.pallas_call(
        body,
        grid=(N // _NB,),
        in_specs=[pl.BlockSpec((2, _NB, D), lambda i: (0, i, 0)),
                  pl.BlockSpec((_NB, D_OUT), lambda i: (i, 0)),
                  pl.BlockSpec((_NB, 1), lambda i: (i, 0)),
                  pl.BlockSpec((1, D_OUT), lambda i: (0, 0))],
        out_specs=pl.BlockSpec((_NB, D_OUT), lambda i: (i, 0)),
        out_shape=jax.ShapeDtypeStruct((N, D_OUT), jnp.float32),
    )(p, g2, dinv, b3)


def kernel(x, edge_index, W1, b1, W2, b2, W3, b3):
    src = edge_index[0]
    dst = edge_index[1]

    deg32 = _sc_degree(dst).reshape(NC * NT, N)
    dinv = lax.rsqrt(jnp.sum(deg32, axis=0) + 1.0)[:, None]
    dinv = jnp.pad(dinv, ((0, NPAD - N), (0, 0)))

    g0 = _tc_prescale(x, dinv)

    pad_src = jnp.arange(EP - E, dtype=src.dtype) % N
    pad_dst = N + (jnp.arange(EP - E, dtype=dst.dtype) % (NPAD - N))
    src_g = jnp.concatenate([src, pad_src]).reshape(G, 8, LANES)
    dst_g = jnp.concatenate([dst, pad_dst]).reshape(G, 8, LANES)
    pack = jnp.concatenate([src_g, dst_g], axis=1).reshape(16 * G, LANES)
    pack_sh = jnp.concatenate([src_g + NPAD, dst_g],
                              axis=1).reshape(16 * G, LANES)
    idx_dup = jnp.stack([pack, pack])
    idx_shift = jnp.stack([pack, pack_sh])

    p0 = _sc_propagate(g0, idx_dup, split=False)
    g1 = _tc_layer1(p0, g0, dinv, W1, b1.reshape(1, -1))
    p1 = _sc_propagate(g1.reshape(2 * NPAD, D), idx_shift, split=True)
    g2 = _tc_layer23(p1, dinv, W2, b2.reshape(1, -1), W3)
    p2 = _sc_propagate(g2, idx_dup, split=False)
    return _tc_final(p2, g2, dinv, b3.reshape(1, -1))
